# baseline (speedup 1.0000x reference)
import functools

import jax
import jax.numpy as jnp
from jax import lax
from jax.experimental import pallas as pl
from jax.experimental.pallas import tpu as pltpu
from jax.experimental.pallas import tpu_sc as plsc

NC = 2
NS = 16
L = 16
ROW = 128
CH = 16


def _sc_two_layer_agg(n_pad, e_rows, feat):
  rt = e_rows // NS
  zrows = n_pad // NS
  ch = CH
  erows_blk = 32
  assert rt % ch == 0 and ch % 8 == 0
  assert zrows % erows_blk == 0
  mesh = plsc.VectorSubcoreMesh(core_axis_name="c", subcore_axis_name="s")

  @functools.partial(
      pl.kernel,
      out_type=[
          jax.ShapeDtypeStruct((2 * n_pad, feat), jnp.float32),
          jax.ShapeDtypeStruct((2 * n_pad, feat), jnp.float32),
      ],
      mesh=mesh,
      scratch_types=[
          pltpu.VMEM_SHARED((n_pad, feat), jnp.float32),
          pltpu.VMEM((ch, ROW), jnp.int32),
          pltpu.VMEM((ch, ROW), jnp.int32),
          pltpu.VMEM((ROW, feat), jnp.float32),
          pltpu.VMEM((ROW, feat), jnp.float32),
          pltpu.VMEM((ROW, feat), jnp.float32),
          pltpu.VMEM((ROW, feat), jnp.float32),
          pltpu.VMEM((erows_blk, feat), jnp.float32),
          pltpu.VMEM((erows_blk, feat), jnp.float32),
          pltpu.VMEM((feat,), jnp.float32),
          pltpu.SemaphoreType.DMA,
          pltpu.SemaphoreType.DMA,
          pltpu.SemaphoreType.DMA,
          pltpu.SemaphoreType.DMA,
      ],
      compiler_params=pltpu.CompilerParams(use_tc_tiling_on_sc=False),
  )
  def k(m2, src_cat, dst2, b1s, x2, agg2, acc, srcb, dstb, rows0, rows1,
        rows2, rows3, zbuf, ebuf, bbuf, sem0, sem1, sem2, sem3):
    c = lax.axis_index("c")
    s = lax.axis_index("s")

    def zb(r, _):
      for kk in range(feat // L):
        zbuf[r, pl.ds(kk * L, L)] = jnp.zeros((L,), jnp.float32)
      return 0
    lax.fori_loop(0, erows_blk, zb, 0)

    zbase = s * zrows
    def zc(i, _):
      pltpu.sync_copy(zbuf, acc.at[pl.ds(zbase + i * erows_blk, erows_blk)])
      return 0
    lax.fori_loop(0, zrows // erows_blk, zc, 0)

    pltpu.sync_copy(b1s.at[c], bbuf)
    plsc.subcore_barrier()

    def agg_pass(table):
      base = s * rt

      def chunk(q, _):
        pltpu.sync_copy(src_cat.at[c, pl.ds(base + q * ch, ch)], srcb)
        pltpu.sync_copy(dst2.at[pl.ds(base + q * ch, ch)], dstb)

        def step(i, _):
          j = 4 * i
          d0 = pltpu.async_copy(table.at[srcb.at[j]], rows0, sem0)
          d1 = pltpu.async_copy(table.at[srcb.at[j + 1]], rows1, sem1)
          d2 = pltpu.async_copy(table.at[srcb.at[j + 2]], rows2, sem2)
          d3 = pltpu.async_copy(table.at[srcb.at[j + 3]], rows3, sem3)
          d0.wait()
          pltpu.sync_copy(rows0, acc.at[dstb.at[j]], add=True)
          d1.wait()
          pltpu.sync_copy(rows1, acc.at[dstb.at[j + 1]], add=True)
          d2.wait()
          pltpu.sync_copy(rows2, acc.at[dstb.at[j + 2]], add=True)
          d3.wait()
          pltpu.sync_copy(rows3, acc.at[dstb.at[j + 3]], add=True)
          return 0
        lax.fori_loop(0, ch // 4, step, 0)
        return 0
      lax.fori_loop(0, rt // ch, chunk, 0)

    agg_pass(m2)
    plsc.subcore_barrier()

    bb = [bbuf[pl.ds(kk * L, L)] for kk in range(feat // L)]
    def ep1(i, _):
      b = zbase + i * erows_blk
      pltpu.sync_copy(acc.at[pl.ds(b, erows_blk)], ebuf)
      pltpu.sync_copy(zbuf, acc.at[pl.ds(b, erows_blk)])
      def rowfix(r, _):
        for kk in range(feat // L):
          v = ebuf[r, pl.ds(kk * L, L)] + bb[kk]
          ebuf[r, pl.ds(kk * L, L)] = jnp.maximum(v, 0.0)
        return 0
      lax.fori_loop(0, erows_blk, rowfix, 0)
      pltpu.sync_copy(ebuf, x2.at[pl.ds(c * n_pad + b, erows_blk)])
      return 0
    lax.fori_loop(0, zrows // erows_blk, ep1, 0)
    plsc.subcore_barrier()

    agg_pass(x2)
    plsc.subcore_barrier()

    def ep2(i, _):
      b = zbase + i * erows_blk
      pltpu.sync_copy(acc.at[pl.ds(b, erows_blk)], ebuf)
      pltpu.sync_copy(ebuf, agg2.at[pl.ds(c * n_pad + b, erows_blk)])
      return 0
    lax.fori_loop(0, zrows // erows_blk, ep2, 0)

  return k


def _tc_matmul1(n_nodes, n_pad, din, dout, feat, bn):
  def body(h_ref, w_ref, o_ref):
    r = jnp.dot(h_ref[...], w_ref[...], preferred_element_type=jnp.float32)
    o_ref[0] = r[:, :feat]
    o_ref[1] = r[:, feat:]

  return pl.pallas_call(
      body,
      grid=(n_nodes // bn,),
      in_specs=[
          pl.BlockSpec((bn, din), lambda i: (i, 0)),
          pl.BlockSpec((din, dout), lambda i: (0, 0)),
      ],
      out_specs=pl.BlockSpec((2, bn, feat), lambda i: (0, i, 0)),
      out_shape=jax.ShapeDtypeStruct((2, n_pad, feat), jnp.float32),
  )


def _tc_matmul2(n_nodes, n_pad, dout, nfeat, feat, bn):
  def body(a_ref, w_ref, b_ref, o_ref):
    a = a_ref[...]
    r = jnp.dot(a[0], w_ref[:feat, :], preferred_element_type=jnp.float32)
    r = r + jnp.dot(a[1], w_ref[feat:, :], preferred_element_type=jnp.float32)
    o_ref[...] = jnp.maximum(r + b_ref[...], 0.0)

  return pl.pallas_call(
      body,
      grid=(n_nodes // bn,),
      in_specs=[
          pl.BlockSpec((2, bn, feat), lambda i: (0, i, 0)),
          pl.BlockSpec((dout, nfeat), lambda i: (0, 0)),
          pl.BlockSpec((nfeat,), lambda i: (0,)),
      ],
      out_specs=pl.BlockSpec((bn, nfeat), lambda i: (i, 0)),
      out_shape=jax.ShapeDtypeStruct((n_nodes, nfeat), jnp.float32),
  )


@jax.jit
def kernel(h, edge_index, W1, b1, W2, b2):
  n_nodes = h.shape[0]
  din = h.shape[1]
  dout = W1.shape[1]
  nfeat = W2.shape[1]
  n_edges = edge_index.shape[1]
  feat = dout // 2

  erow_pad = -(-n_edges // (ROW * NS * CH)) * (NS * CH)
  e_pad = erow_pad * ROW
  n_pad = -(-(n_nodes + 1) // (32 * NS)) * (32 * NS)

  src = edge_index[0]
  dst = edge_index[1]
  pad = e_pad - n_edges
  src_p = jnp.concatenate([src, jnp.zeros((pad,), jnp.int32)])
  dst_p = jnp.concatenate([dst, jnp.full((pad,), n_nodes, jnp.int32)])
  src2 = src_p.reshape(erow_pad, ROW)
  src_cat = jnp.stack([src2, src2 + n_pad])
  dst2 = dst_p.reshape(erow_pad, ROW)
  b1s = b1.reshape(2, feat)

  m3 = _tc_matmul1(n_nodes, n_pad, din, dout, feat, 2000)(h, W1)
  m2 = m3.reshape(2 * n_pad, feat)

  sc = _sc_two_layer_agg(n_pad, erow_pad, feat)
  _, agg2 = sc(m2, src_cat, dst2, b1s)

  agg2r = agg2.reshape(2, n_pad, feat)
  return _tc_matmul2(n_nodes, n_pad, dout, nfeat, feat, 2000)(agg2r, W2, b2)


# speedup vs baseline: 1.5271x; 1.5271x over previous
import functools

import jax
import jax.numpy as jnp
from jax import lax
from jax.experimental import pallas as pl
from jax.experimental.pallas import tpu as pltpu
from jax.experimental.pallas import tpu_sc as plsc

NC = 2
NS = 16
L = 16
ROW = 128
CH = 8


def _sc_two_layer_agg(n_pad, e_rows, feat):
  rt = e_rows // NS
  zrows = n_pad // NS
  ch = CH
  erows_blk = 64
  assert rt % ch == 0 and ch % 8 == 0
  assert zrows % erows_blk == 0
  mesh = plsc.VectorSubcoreMesh(core_axis_name="c", subcore_axis_name="s")

  @functools.partial(
      pl.kernel,
      out_type=[
          jax.ShapeDtypeStruct((2 * n_pad, feat), jnp.float32),
          jax.ShapeDtypeStruct((2 * n_pad, feat), jnp.float32),
      ],
      mesh=mesh,
      scratch_types=[
          pltpu.VMEM_SHARED((n_pad, feat), jnp.float32),
          pltpu.VMEM((ch, ROW), jnp.int32),
          pltpu.VMEM((ch, ROW), jnp.int32),
          pltpu.VMEM((ROW, feat), jnp.float32),
          pltpu.VMEM((ROW, feat), jnp.float32),
          pltpu.VMEM((ROW, feat), jnp.float32),
          pltpu.VMEM((ROW, feat), jnp.float32),
          pltpu.VMEM((erows_blk, feat), jnp.float32),
          pltpu.VMEM((erows_blk, feat), jnp.float32),
          pltpu.VMEM((feat,), jnp.float32),
          pltpu.SemaphoreType.DMA,
          pltpu.SemaphoreType.DMA,
          pltpu.SemaphoreType.DMA,
          pltpu.SemaphoreType.DMA,
      ],
      compiler_params=pltpu.CompilerParams(use_tc_tiling_on_sc=False),
  )
  def k(m2, src_cat, dst2, b1s, x2, agg2, acc, srcb, dstb, rows0, rows1,
        rows2, rows3, zbuf, ebuf, bbuf, sem0, sem1, sem2, sem3):
    c = lax.axis_index("c")
    s = lax.axis_index("s")

    def zb(r, _):
      for kk in range(feat // L):
        zbuf[r, pl.ds(kk * L, L)] = jnp.zeros((L,), jnp.float32)
      return 0
    lax.fori_loop(0, erows_blk, zb, 0)

    zbase = s * zrows
    def zc(i, _):
      pltpu.sync_copy(zbuf, acc.at[pl.ds(zbase + i * erows_blk, erows_blk)])
      return 0
    lax.fori_loop(0, zrows // erows_blk, zc, 0)

    pltpu.sync_copy(b1s.at[c], bbuf)
    plsc.subcore_barrier()

    def agg_pass(table):
      base = s * rt

      def chunk(q, _):
        pltpu.sync_copy(src_cat.at[c, pl.ds(base + q * ch, ch)], srcb)
        pltpu.sync_copy(dst2.at[pl.ds(base + q * ch, ch)], dstb)

        def step(i, _):
          j = 4 * i
          d0 = pltpu.async_copy(table.at[srcb.at[j]], rows0, sem0)
          d1 = pltpu.async_copy(table.at[srcb.at[j + 1]], rows1, sem1)
          d2 = pltpu.async_copy(table.at[srcb.at[j + 2]], rows2, sem2)
          d3 = pltpu.async_copy(table.at[srcb.at[j + 3]], rows3, sem3)
          d0.wait()
          pltpu.sync_copy(rows0, acc.at[dstb.at[j]], add=True)
          d1.wait()
          pltpu.sync_copy(rows1, acc.at[dstb.at[j + 1]], add=True)
          d2.wait()
          pltpu.sync_copy(rows2, acc.at[dstb.at[j + 2]], add=True)
          d3.wait()
          pltpu.sync_copy(rows3, acc.at[dstb.at[j + 3]], add=True)
          return 0
        lax.fori_loop(0, ch // 4, step, 0)
        return 0
      lax.fori_loop(0, rt // ch, chunk, 0)

    agg_pass(m2)
    plsc.subcore_barrier()

    bb = [bbuf[pl.ds(kk * L, L)] for kk in range(feat // L)]
    def ep1(i, _):
      b = zbase + i * erows_blk
      pltpu.sync_copy(acc.at[pl.ds(b, erows_blk)], ebuf)
      pltpu.sync_copy(zbuf, acc.at[pl.ds(b, erows_blk)])
      def rowfix(r, _):
        for kk in range(feat // L):
          v = ebuf[r, pl.ds(kk * L, L)] + bb[kk]
          ebuf[r, pl.ds(kk * L, L)] = jnp.maximum(v, 0.0)
        return 0
      lax.fori_loop(0, erows_blk, rowfix, 0)
      pltpu.sync_copy(ebuf, x2.at[pl.ds(c * n_pad + b, erows_blk)])
      return 0
    lax.fori_loop(0, zrows // erows_blk, ep1, 0)
    plsc.subcore_barrier()

    agg_pass(x2)
    plsc.subcore_barrier()

    def ep2(i, _):
      b = zbase + i * erows_blk
      pltpu.sync_copy(acc.at[pl.ds(b, erows_blk)], ebuf)
      pltpu.sync_copy(ebuf, agg2.at[pl.ds(c * n_pad + b, erows_blk)])
      return 0
    lax.fori_loop(0, zrows // erows_blk, ep2, 0)

  return k


def _tc_matmul1(n_nodes, n_pad, din, dout, feat, bn):
  def body(h_ref, w_ref, o_ref):
    r = jnp.dot(h_ref[...], w_ref[...], preferred_element_type=jnp.float32)
    o_ref[0] = r[:, :feat]
    o_ref[1] = r[:, feat:]

  return pl.pallas_call(
      body,
      grid=(n_nodes // bn,),
      in_specs=[
          pl.BlockSpec((bn, din), lambda i: (i, 0)),
          pl.BlockSpec((din, dout), lambda i: (0, 0)),
      ],
      out_specs=pl.BlockSpec((2, bn, feat), lambda i: (0, i, 0)),
      out_shape=jax.ShapeDtypeStruct((2, n_pad, feat), jnp.float32),
  )


def _tc_matmul2(n_nodes, n_pad, dout, nfeat, feat, bn):
  def body(a_ref, w_ref, b_ref, o_ref):
    a = a_ref[...]
    r = jnp.dot(a[0], w_ref[:feat, :], preferred_element_type=jnp.float32)
    r = r + jnp.dot(a[1], w_ref[feat:, :], preferred_element_type=jnp.float32)
    o_ref[...] = jnp.maximum(r + b_ref[...], 0.0)

  return pl.pallas_call(
      body,
      grid=(n_nodes // bn,),
      in_specs=[
          pl.BlockSpec((2, bn, feat), lambda i: (0, i, 0)),
          pl.BlockSpec((dout, nfeat), lambda i: (0, 0)),
          pl.BlockSpec((nfeat,), lambda i: (0,)),
      ],
      out_specs=pl.BlockSpec((bn, nfeat), lambda i: (i, 0)),
      out_shape=jax.ShapeDtypeStruct((n_nodes, nfeat), jnp.float32),
  )


@jax.jit
def kernel(h, edge_index, W1, b1, W2, b2):
  n_nodes = h.shape[0]
  din = h.shape[1]
  dout = W1.shape[1]
  nfeat = W2.shape[1]
  n_edges = edge_index.shape[1]
  feat = dout // 2

  erow_pad = -(-n_edges // (ROW * NS * CH)) * (NS * CH)
  e_pad = erow_pad * ROW
  n_pad = -(-(n_nodes + 1) // (128 * NS)) * (128 * NS)

  src = edge_index[0]
  dst = edge_index[1]
  pad = e_pad - n_edges
  src_p = jnp.concatenate([src, jnp.zeros((pad,), jnp.int32)])
  dst_p = jnp.concatenate([dst, jnp.full((pad,), n_nodes, jnp.int32)])
  src2 = src_p.reshape(erow_pad, ROW)
  src_cat = jnp.stack([src2, src2 + n_pad])
  dst2 = dst_p.reshape(erow_pad, ROW)
  b1s = b1.reshape(2, feat)

  m3 = _tc_matmul1(n_nodes, n_pad, din, dout, feat, 2000)(h, W1)
  m2 = m3.reshape(2 * n_pad, feat)

  sc = _sc_two_layer_agg(n_pad, erow_pad, feat)
  _, agg2 = sc(m2, src_cat, dst2, b1s)

  agg2r = agg2.reshape(2, n_pad, feat)
  return _tc_matmul2(n_nodes, n_pad, dout, nfeat, feat, 2000)(agg2r, W2, b2)


# speedup vs baseline: 1.5383x; 1.0074x over previous
import functools

import jax
import jax.numpy as jnp
from jax import lax
from jax.experimental import pallas as pl
from jax.experimental.pallas import tpu as pltpu
from jax.experimental.pallas import tpu_sc as plsc

NC = 2
NS = 16
L = 16
ROW = 128
CH = 8


def _sc_two_layer_agg(n_pad, e_rows, feat):
  rt = e_rows // NS
  zrows = n_pad // NS
  ch = CH
  erows_blk = 128
  assert rt % ch == 0 and ch % 8 == 0
  assert zrows % erows_blk == 0
  mesh = plsc.VectorSubcoreMesh(core_axis_name="c", subcore_axis_name="s")

  @functools.partial(
      pl.kernel,
      out_type=[
          jax.ShapeDtypeStruct((2 * n_pad, feat), jnp.float32),
          jax.ShapeDtypeStruct((2 * n_pad, feat), jnp.float32),
      ],
      mesh=mesh,
      scratch_types=[
          pltpu.VMEM_SHARED((n_pad, feat), jnp.float32),
          pltpu.VMEM((ch, ROW), jnp.int32),
          pltpu.VMEM((ch, ROW), jnp.int32),
          pltpu.VMEM((ROW, feat), jnp.float32),
          pltpu.VMEM((ROW, feat), jnp.float32),
          pltpu.VMEM((ROW, feat), jnp.float32),
          pltpu.VMEM((ROW, feat), jnp.float32),
          pltpu.VMEM((erows_blk, feat), jnp.float32),
          pltpu.VMEM((erows_blk, feat), jnp.float32),
          pltpu.VMEM((feat,), jnp.float32),
          pltpu.SemaphoreType.DMA,
          pltpu.SemaphoreType.DMA,
          pltpu.SemaphoreType.DMA,
          pltpu.SemaphoreType.DMA,
      ],
      compiler_params=pltpu.CompilerParams(use_tc_tiling_on_sc=False),
  )
  def k(m2, src_cat, dst2, b1s, x2, agg2, acc, srcb, dstb, rows0, rows1,
        rows2, rows3, zbuf, ebuf, bbuf, sem0, sem1, sem2, sem3):
    c = lax.axis_index("c")
    s = lax.axis_index("s")

    def zb(r, _):
      for kk in range(feat // L):
        zbuf[r, pl.ds(kk * L, L)] = jnp.zeros((L,), jnp.float32)
      return 0
    lax.fori_loop(0, erows_blk, zb, 0)

    zbase = s * zrows
    def zc(i, _):
      pltpu.sync_copy(zbuf, acc.at[pl.ds(zbase + i * erows_blk, erows_blk)])
      return 0
    lax.fori_loop(0, zrows // erows_blk, zc, 0)

    pltpu.sync_copy(b1s.at[c], bbuf)
    plsc.subcore_barrier()

    def agg_pass(table):
      base = s * rt

      def chunk(q, _):
        pltpu.sync_copy(src_cat.at[c, pl.ds(base + q * ch, ch)], srcb)
        pltpu.sync_copy(dst2.at[pl.ds(base + q * ch, ch)], dstb)

        def step(i, _):
          j = 4 * i
          d0 = pltpu.async_copy(table.at[srcb.at[j]], rows0, sem0)
          d1 = pltpu.async_copy(table.at[srcb.at[j + 1]], rows1, sem1)
          d2 = pltpu.async_copy(table.at[srcb.at[j + 2]], rows2, sem2)
          d3 = pltpu.async_copy(table.at[srcb.at[j + 3]], rows3, sem3)
          d0.wait()
          pltpu.sync_copy(rows0, acc.at[dstb.at[j]], add=True)
          d1.wait()
          pltpu.sync_copy(rows1, acc.at[dstb.at[j + 1]], add=True)
          d2.wait()
          pltpu.sync_copy(rows2, acc.at[dstb.at[j + 2]], add=True)
          d3.wait()
          pltpu.sync_copy(rows3, acc.at[dstb.at[j + 3]], add=True)
          return 0
        lax.fori_loop(0, ch // 4, step, 0)
        return 0
      lax.fori_loop(0, rt // ch, chunk, 0)

    agg_pass(m2)
    plsc.subcore_barrier()

    bb = [bbuf[pl.ds(kk * L, L)] for kk in range(feat // L)]
    def ep1(i, _):
      b = zbase + i * erows_blk
      pltpu.sync_copy(acc.at[pl.ds(b, erows_blk)], ebuf)
      pltpu.sync_copy(zbuf, acc.at[pl.ds(b, erows_blk)])
      def rowfix(r, _):
        for kk in range(feat // L):
          v = ebuf[r, pl.ds(kk * L, L)] + bb[kk]
          ebuf[r, pl.ds(kk * L, L)] = jnp.maximum(v, 0.0)
        return 0
      lax.fori_loop(0, erows_blk, rowfix, 0)
      pltpu.sync_copy(ebuf, x2.at[pl.ds(c * n_pad + b, erows_blk)])
      return 0
    lax.fori_loop(0, zrows // erows_blk, ep1, 0)
    plsc.subcore_barrier()

    agg_pass(x2)
    plsc.subcore_barrier()

    def ep2(i, _):
      b = zbase + i * erows_blk
      pltpu.sync_copy(acc.at[pl.ds(b, erows_blk)],
                      agg2.at[pl.ds(c * n_pad + b, erows_blk)])
      return 0
    lax.fori_loop(0, zrows // erows_blk, ep2, 0)

  return k


def _tc_matmul1(n_nodes, n_pad, din, dout, feat, bn):
  def body(h_ref, w_ref, o_ref):
    r = jnp.dot(h_ref[...], w_ref[...], preferred_element_type=jnp.float32)
    o_ref[0] = r[:, :feat]
    o_ref[1] = r[:, feat:]

  return pl.pallas_call(
      body,
      grid=(n_nodes // bn,),
      in_specs=[
          pl.BlockSpec((bn, din), lambda i: (i, 0)),
          pl.BlockSpec((din, dout), lambda i: (0, 0)),
      ],
      out_specs=pl.BlockSpec((2, bn, feat), lambda i: (0, i, 0)),
      out_shape=jax.ShapeDtypeStruct((2, n_pad, feat), jnp.float32),
  )


def _tc_matmul2(n_nodes, n_pad, dout, nfeat, feat, bn):
  def body(a_ref, w_ref, b_ref, o_ref):
    a = a_ref[...]
    r = jnp.dot(a[0], w_ref[:feat, :], preferred_element_type=jnp.float32)
    r = r + jnp.dot(a[1], w_ref[feat:, :], preferred_element_type=jnp.float32)
    o_ref[...] = jnp.maximum(r + b_ref[...], 0.0)

  return pl.pallas_call(
      body,
      grid=(n_nodes // bn,),
      in_specs=[
          pl.BlockSpec((2, bn, feat), lambda i: (0, i, 0)),
          pl.BlockSpec((dout, nfeat), lambda i: (0, 0)),
          pl.BlockSpec((nfeat,), lambda i: (0,)),
      ],
      out_specs=pl.BlockSpec((bn, nfeat), lambda i: (i, 0)),
      out_shape=jax.ShapeDtypeStruct((n_nodes, nfeat), jnp.float32),
  )


@jax.jit
def kernel(h, edge_index, W1, b1, W2, b2):
  n_nodes = h.shape[0]
  din = h.shape[1]
  dout = W1.shape[1]
  nfeat = W2.shape[1]
  n_edges = edge_index.shape[1]
  feat = dout // 2

  erow_pad = -(-n_edges // (ROW * NS * CH)) * (NS * CH)
  e_pad = erow_pad * ROW
  n_pad = -(-(n_nodes + 1) // (128 * NS)) * (128 * NS)

  src = edge_index[0]
  dst = edge_index[1]
  pad = e_pad - n_edges
  src_p = jnp.concatenate([src, jnp.zeros((pad,), jnp.int32)])
  dst_p = jnp.concatenate([dst, jnp.full((pad,), n_nodes, jnp.int32)])
  src2 = src_p.reshape(erow_pad, ROW)
  src_cat = jnp.stack([src2, src2 + n_pad])
  dst2 = dst_p.reshape(erow_pad, ROW)
  b1s = b1.reshape(2, feat)

  m3 = _tc_matmul1(n_nodes, n_pad, din, dout, feat, 2000)(h, W1)
  m2 = m3.reshape(2 * n_pad, feat)

  sc = _sc_two_layer_agg(n_pad, erow_pad, feat)
  _, agg2 = sc(m2, src_cat, dst2, b1s)

  agg2r = agg2.reshape(2, n_pad, feat)
  return _tc_matmul2(n_nodes, n_pad, dout, nfeat, feat, 2000)(agg2r, W2, b2)


# speedup vs baseline: 1.6068x; 1.0445x over previous
import functools

import jax
import jax.numpy as jnp
from jax import lax
from jax.experimental import pallas as pl
from jax.experimental.pallas import tpu as pltpu
from jax.experimental.pallas import tpu_sc as plsc

NC = 2
NS = 16
L = 16
ROW = 128
CH = 8


def _sc_two_layer_agg(n_pad, e_rows, feat):
  rt = e_rows // NS
  zrows = n_pad // NS
  ch = CH
  erows_blk = 128
  assert rt % ch == 0 and ch % 8 == 0
  assert zrows % erows_blk == 0
  mesh = plsc.VectorSubcoreMesh(core_axis_name="c", subcore_axis_name="s")

  @functools.partial(
      pl.kernel,
      out_type=[
          jax.ShapeDtypeStruct((2 * n_pad, feat), jnp.float32),
          jax.ShapeDtypeStruct((2 * n_pad, feat), jnp.float32),
      ],
      mesh=mesh,
      scratch_types=[
          pltpu.VMEM_SHARED((n_pad, feat), jnp.float32),
          pltpu.VMEM((ch, ROW), jnp.int32),
          pltpu.VMEM((ch, ROW), jnp.int32),
          pltpu.VMEM((ROW, feat), jnp.float32),
          pltpu.VMEM((ROW, feat), jnp.float32),
          pltpu.VMEM((ROW, feat), jnp.float32),
          pltpu.VMEM((ROW, feat), jnp.float32),
          pltpu.VMEM((erows_blk, feat), jnp.float32),
          pltpu.VMEM((erows_blk, feat), jnp.float32),
          pltpu.VMEM((feat,), jnp.float32),
          pltpu.SemaphoreType.DMA,
          pltpu.SemaphoreType.DMA,
          pltpu.SemaphoreType.DMA,
          pltpu.SemaphoreType.DMA,
          pltpu.SemaphoreType.DMA,
          pltpu.SemaphoreType.DMA,
          pltpu.SemaphoreType.DMA,
          pltpu.SemaphoreType.DMA,
      ],
      compiler_params=pltpu.CompilerParams(use_tc_tiling_on_sc=False),
  )
  def k(m2, src_cat, dst2, b1s, x2, agg2, acc, srcb, dstb, rows0, rows1,
        rows2, rows3, zbuf, ebuf, bbuf, sem0, sem1, sem2, sem3,
        ssem0, ssem1, ssem2, ssem3):
    c = lax.axis_index("c")
    s = lax.axis_index("s")

    def zb(r, _):
      for kk in range(feat // L):
        zbuf[r, pl.ds(kk * L, L)] = jnp.zeros((L,), jnp.float32)
      return 0
    lax.fori_loop(0, erows_blk, zb, 0)

    zbase = s * zrows
    def zc(i, _):
      pltpu.sync_copy(zbuf, acc.at[pl.ds(zbase + i * erows_blk, erows_blk)])
      return 0
    lax.fori_loop(0, zrows // erows_blk, zc, 0)

    pltpu.sync_copy(b1s.at[c], bbuf)
    plsc.subcore_barrier()

    def agg_pass(table):
      base = s * rt

      def chunk(q, _):
        pltpu.sync_copy(src_cat.at[c, pl.ds(base + q * ch, ch)], srcb)
        pltpu.sync_copy(dst2.at[pl.ds(base + q * ch, ch)], dstb)

        def step(i, _):
          j = 4 * i
          d0 = pltpu.async_copy(table.at[srcb.at[j]], rows0, sem0)
          d1 = pltpu.async_copy(table.at[srcb.at[j + 1]], rows1, sem1)
          d2 = pltpu.async_copy(table.at[srcb.at[j + 2]], rows2, sem2)
          d3 = pltpu.async_copy(table.at[srcb.at[j + 3]], rows3, sem3)
          d0.wait()
          s0 = pltpu.async_copy(rows0, acc.at[dstb.at[j]], ssem0, add=True)
          d1.wait()
          s1 = pltpu.async_copy(rows1, acc.at[dstb.at[j + 1]], ssem1,
                                add=True)
          d2.wait()
          s2 = pltpu.async_copy(rows2, acc.at[dstb.at[j + 2]], ssem2,
                                add=True)
          d3.wait()
          s3 = pltpu.async_copy(rows3, acc.at[dstb.at[j + 3]], ssem3,
                                add=True)
          s0.wait(); s1.wait(); s2.wait(); s3.wait()
          return 0
        lax.fori_loop(0, ch // 4, step, 0)
        return 0
      lax.fori_loop(0, rt // ch, chunk, 0)

    agg_pass(m2)
    plsc.subcore_barrier()

    bb = [bbuf[pl.ds(kk * L, L)] for kk in range(feat // L)]
    def ep1(i, _):
      b = zbase + i * erows_blk
      pltpu.sync_copy(acc.at[pl.ds(b, erows_blk)], ebuf)
      pltpu.sync_copy(zbuf, acc.at[pl.ds(b, erows_blk)])
      def rowfix(r, _):
        for kk in range(feat // L):
          v = ebuf[r, pl.ds(kk * L, L)] + bb[kk]
          ebuf[r, pl.ds(kk * L, L)] = jnp.maximum(v, 0.0)
        return 0
      lax.fori_loop(0, erows_blk, rowfix, 0)
      pltpu.sync_copy(ebuf, x2.at[pl.ds(c * n_pad + b, erows_blk)])
      return 0
    lax.fori_loop(0, zrows // erows_blk, ep1, 0)
    plsc.subcore_barrier()

    agg_pass(x2)
    plsc.subcore_barrier()

    def ep2(i, _):
      b = zbase + i * erows_blk
      pltpu.sync_copy(acc.at[pl.ds(b, erows_blk)],
                      agg2.at[pl.ds(c * n_pad + b, erows_blk)])
      return 0
    lax.fori_loop(0, zrows // erows_blk, ep2, 0)

  return k


def _tc_matmul1(n_nodes, n_pad, din, dout, feat, bn):
  def body(h_ref, w_ref, o_ref):
    r = jnp.dot(h_ref[...], w_ref[...], preferred_element_type=jnp.float32)
    o_ref[0] = r[:, :feat]
    o_ref[1] = r[:, feat:]

  return pl.pallas_call(
      body,
      grid=(n_nodes // bn,),
      in_specs=[
          pl.BlockSpec((bn, din), lambda i: (i, 0)),
          pl.BlockSpec((din, dout), lambda i: (0, 0)),
      ],
      out_specs=pl.BlockSpec((2, bn, feat), lambda i: (0, i, 0)),
      out_shape=jax.ShapeDtypeStruct((2, n_pad, feat), jnp.float32),
  )


def _tc_matmul2(n_nodes, n_pad, dout, nfeat, feat, bn):
  def body(a_ref, w_ref, b_ref, o_ref):
    a = a_ref[...]
    r = jnp.dot(a[0], w_ref[:feat, :], preferred_element_type=jnp.float32)
    r = r + jnp.dot(a[1], w_ref[feat:, :], preferred_element_type=jnp.float32)
    o_ref[...] = jnp.maximum(r + b_ref[...], 0.0)

  return pl.pallas_call(
      body,
      grid=(n_nodes // bn,),
      in_specs=[
          pl.BlockSpec((2, bn, feat), lambda i: (0, i, 0)),
          pl.BlockSpec((dout, nfeat), lambda i: (0, 0)),
          pl.BlockSpec((nfeat,), lambda i: (0,)),
      ],
      out_specs=pl.BlockSpec((bn, nfeat), lambda i: (i, 0)),
      out_shape=jax.ShapeDtypeStruct((n_nodes, nfeat), jnp.float32),
  )


@jax.jit
def kernel(h, edge_index, W1, b1, W2, b2):
  n_nodes = h.shape[0]
  din = h.shape[1]
  dout = W1.shape[1]
  nfeat = W2.shape[1]
  n_edges = edge_index.shape[1]
  feat = dout // 2

  erow_pad = -(-n_edges // (ROW * NS * CH)) * (NS * CH)
  e_pad = erow_pad * ROW
  n_pad = -(-(n_nodes + 1) // (128 * NS)) * (128 * NS)

  src = edge_index[0]
  dst = edge_index[1]
  pad = e_pad - n_edges
  src_p = jnp.concatenate([src, jnp.zeros((pad,), jnp.int32)])
  dst_p = jnp.concatenate([dst, jnp.full((pad,), n_nodes, jnp.int32)])
  src2 = src_p.reshape(erow_pad, ROW)
  src_cat = jnp.stack([src2, src2 + n_pad])
  dst2 = dst_p.reshape(erow_pad, ROW)
  b1s = b1.reshape(2, feat)

  m3 = _tc_matmul1(n_nodes, n_pad, din, dout, feat, 2000)(h, W1)
  m2 = m3.reshape(2 * n_pad, feat)

  sc = _sc_two_layer_agg(n_pad, erow_pad, feat)
  _, agg2 = sc(m2, src_cat, dst2, b1s)

  agg2r = agg2.reshape(2, n_pad, feat)
  return _tc_matmul2(n_nodes, n_pad, dout, nfeat, feat, 2000)(agg2r, W2, b2)
